# baseline (speedup 1.0000x reference)
import functools

import jax
import jax.numpy as jnp
from jax import lax
from jax.experimental import pallas as pl
from jax.experimental.pallas import tpu as pltpu
from jax.experimental.pallas import tpu_sc as plsc

DIM = 32
OUT_DIM = 64
B = 4096
T = 50
NROW = 1000000
NQUAD = 262144
OFF3 = 741376
K3_MIN = 786432

NC, NS = 2, 16
NW = NC * NS
NBLK = B // NW
L = 16

_sc_mesh = plsc.VectorSubcoreMesh(core_axis_name="c", subcore_axis_name="s")


@functools.partial(
    pl.kernel,
    out_type=jax.ShapeDtypeStruct((T, B, DIM), jnp.float32),
    mesh=_sc_mesh,
    scratch_types=[
        pltpu.VMEM((T, NBLK), jnp.int32),
        pltpu.VMEM((T, NBLK), jnp.int32),
        pltpu.VMEM((T, NBLK), jnp.int32),
        pltpu.VMEM((2, NBLK, 128), jnp.float32),
        pltpu.VMEM((2, NBLK, DIM), jnp.float32),
        pltpu.SemaphoreType.DMA,
        pltpu.SemaphoreType.DMA,
        pltpu.SemaphoreType.DMA,
        pltpu.SemaphoreType.DMA,
    ],
    compiler_params=pltpu.CompilerParams(use_tc_tiling_on_sc=True),
)
def _sc_gather(tokT_hbm, table_hbm, inter_hbm, tok_v, q_v, r_v, buf_v,
               slab_v, g0, g1, w0, w1):
    wid = lax.axis_index("s") * NC + lax.axis_index("c")
    n0 = wid * NBLK

    pltpu.sync_copy(tokT_hbm.at[:, pl.ds(n0, NBLK)], tok_v)

    def _pre(t, _):
        for g in range(NBLK // L):
            sl = pl.ds(g * L, L)
            tok = tok_v[t, sl]
            hi = tok >= K3_MIN
            k = jnp.where(hi, 3, lax.shift_right_logical(tok, 18))
            q_v[t, sl] = jnp.where(hi, tok - OFF3,
                                   lax.bitwise_and(tok, NQUAD - 1))
            r_v[t, sl] = lax.shift_left(k, 5)
        return 0

    lax.fori_loop(0, T, _pre, 0, unroll=False)

    gsems = (g0, g1)
    wsems = (w0, w1)

    def _gather(i, par):
        return pltpu.make_async_copy(
            table_hbm.at[q_v.at[i]], buf_v.at[par], gsems[par])

    def _slab_copy(i, par):
        return pltpu.make_async_copy(
            slab_v.at[par], inter_hbm.at[i, pl.ds(n0, NBLK), :], wsems[par])

    def _step(i, par):
        @pl.when(i + 1 < T)
        def _():
            _gather(i + 1, 1 - par).start()

        _gather(i, par).wait()

        @pl.when(i >= 2)
        def _():
            _slab_copy(i, par).wait()

        buf = buf_v.at[par]
        slab = slab_v.at[par]
        for g in range(NBLK // L):
            rsv = r_v[i, pl.ds(g * L, L)]
            for l in range(L):
                rs = rsv[l]
                j = g * L + l
                slab[j, pl.ds(0, L)] = buf[j, pl.ds(rs, L)]
                slab[j, pl.ds(L, L)] = buf[j, pl.ds(rs + L, L)]
        _slab_copy(i, par).start()
        return 0

    def _pair(k, _):
        _step(2 * k, 0)
        _step(2 * k + 1, 1)
        return 0

    _gather(0, 0).start()
    lax.fori_loop(0, T // 2, _pair, 0, unroll=False)

    _slab_copy(T - 2, 0).wait()
    _slab_copy(T - 1, 1).wait()


RPW = 4096
RPG = NQUAD // RPW
_KOFF = (0, NQUAD // RPW, 2 * NQUAD // RPW, OFF3 // RPW)


def _repack_body(x0_ref, x1_ref, x2_ref, x3_ref, out_ref):
    ident = jnp.eye(DIM, dtype=jnp.float32)
    for k, x_ref in enumerate((x0_ref, x1_ref, x2_ref, x3_ref)):
        out_ref[:, k * DIM:(k + 1) * DIM] = lax.dot_general(
            x_ref[...], ident,
            dimension_numbers=(((0,), (0,)), ((), ())),
            preferred_element_type=jnp.float32,
        )


_repack = pl.pallas_call(
    _repack_body,
    grid=(RPG,),
    in_specs=[
        pl.BlockSpec((DIM, RPW), lambda j, o=o: (0, o + j)) for o in _KOFF
    ],
    out_specs=pl.BlockSpec((RPW, 128), lambda j: (j, 0)),
    out_shape=jax.ShapeDtypeStruct((NQUAD, 128), jnp.float32),
)


def _proj_body(g_ref, m_ref, w_ref, b_ref, out_ref):
    g = g_ref[0]
    acc = lax.dot_general(
        w_ref[...], g,
        dimension_numbers=(((1,), (1,)), ((), ())),
        preferred_element_type=jnp.float32,
    )
    out_ref[0] = (acc + b_ref[...]) * m_ref[0]


_proj = pl.pallas_call(
    _proj_body,
    grid=(T,),
    in_specs=[
        pl.BlockSpec((1, B, DIM), lambda t: (t, 0, 0)),
        pl.BlockSpec((1, 1, B), lambda t: (t, 0, 0)),
        pl.BlockSpec((OUT_DIM, DIM), lambda t: (0, 0)),
        pl.BlockSpec((OUT_DIM, 1), lambda t: (0, 0)),
    ],
    out_specs=pl.BlockSpec((1, OUT_DIM, B), lambda t: (t, 0, 0)),
    out_shape=jax.ShapeDtypeStruct((T, OUT_DIM, B), jnp.float32),
)


def kernel(tokens, mask, embed_table, W, b):
    tokT = tokens.T.astype(jnp.int32)
    tT = embed_table.T
    table4 = _repack(tT, tT, tT, tT)
    inter = _sc_gather(tokT, table4)
    maskT = mask.T.astype(jnp.float32).reshape(T, 1, B)
    out = _proj(inter, maskT, W, b.reshape(OUT_DIM, 1))
    return jnp.transpose(out, (2, 0, 1)), mask


# speedup vs baseline: 1.5611x; 1.5611x over previous
import functools

import jax
import jax.numpy as jnp
from jax import lax
from jax.experimental import pallas as pl
from jax.experimental.pallas import tpu as pltpu
from jax.experimental.pallas import tpu_sc as plsc

DIM = 32
OUT_DIM = 64
B = 4096
T = 50
NROW = 1000000
NQUAD = 262144
OFF3 = 741376
K3_MIN = 786432

NC, NS = 2, 16
NW = NC * NS
NBLK = B // NW
L = 16

_sc_mesh = plsc.VectorSubcoreMesh(core_axis_name="c", subcore_axis_name="s")


@functools.partial(
    pl.kernel,
    out_type=jax.ShapeDtypeStruct((T, B, DIM), jnp.float32),
    mesh=_sc_mesh,
    scratch_types=[
        pltpu.VMEM((T, NBLK), jnp.int32),
        pltpu.VMEM((T, NBLK), jnp.int32),
        pltpu.VMEM((T, NBLK), jnp.int32),
        pltpu.VMEM((2, NBLK, 128), jnp.float32),
        pltpu.VMEM((2, NBLK, DIM), jnp.float32),
        pltpu.SemaphoreType.DMA,
        pltpu.SemaphoreType.DMA,
        pltpu.SemaphoreType.DMA,
        pltpu.SemaphoreType.DMA,
    ],
    compiler_params=pltpu.CompilerParams(use_tc_tiling_on_sc=True),
)
def _sc_gather(tokT_hbm, table_hbm, inter_hbm, tok_v, q_v, r_v, buf_v,
               slab_v, g0, g1, w0, w1):
    wid = lax.axis_index("s") * NC + lax.axis_index("c")
    n0 = wid * NBLK

    pltpu.sync_copy(tokT_hbm.at[:, pl.ds(n0, NBLK)], tok_v)

    def _pre(t, _):
        for g in range(NBLK // L):
            sl = pl.ds(g * L, L)
            tok = tok_v[t, sl]
            hi = tok >= K3_MIN
            k = jnp.where(hi, 3, lax.shift_right_logical(tok, 18))
            q_v[t, sl] = jnp.where(hi, tok - OFF3,
                                   lax.bitwise_and(tok, NQUAD - 1))
            r_v[t, sl] = lax.shift_left(k, 5)
        return 0

    lax.fori_loop(0, T, _pre, 0, unroll=False)

    gsems = (g0, g1)
    wsems = (w0, w1)

    def _gather(i, par):
        return pltpu.make_async_copy(
            table_hbm.at[q_v.at[i]], buf_v.at[par], gsems[par])

    def _slab_copy(i, par):
        return pltpu.make_async_copy(
            slab_v.at[par], inter_hbm.at[i, pl.ds(n0, NBLK), :], wsems[par])

    def _step(i, par):
        @pl.when(i + 1 < T)
        def _():
            _gather(i + 1, 1 - par).start()

        _gather(i, par).wait()

        @pl.when(i >= 2)
        def _():
            _slab_copy(i, par).wait()

        buf = buf_v.at[par]
        slab = slab_v.at[par]
        for g in range(NBLK // L):
            rsv = r_v[i, pl.ds(g * L, L)]
            for l in range(L):
                rs = rsv[l]
                j = g * L + l
                slab[j, pl.ds(0, L)] = buf[j, pl.ds(rs, L)]
                slab[j, pl.ds(L, L)] = buf[j, pl.ds(rs + L, L)]
        _slab_copy(i, par).start()
        return 0

    def _pair(k, _):
        _step(2 * k, 0)
        _step(2 * k + 1, 1)
        return 0

    _gather(0, 0).start()
    lax.fori_loop(0, T // 2, _pair, 0, unroll=False)

    _slab_copy(T - 2, 0).wait()
    _slab_copy(T - 1, 1).wait()


RPW = 4096
RPG = NQUAD // RPW
_KOFF = (0, NQUAD // RPW, 2 * NQUAD // RPW, OFF3 // RPW)


def _repack_body(x0_ref, x1_ref, x2_ref, x3_ref, out_ref):
    x4 = jnp.concatenate(
        [x0_ref[...], x1_ref[...], x2_ref[...], x3_ref[...]], axis=0)
    out_ref[...] = x4.T


_repack = pl.pallas_call(
    _repack_body,
    grid=(RPG,),
    in_specs=[
        pl.BlockSpec((DIM, RPW), lambda j, o=o: (0, o + j)) for o in _KOFF
    ],
    out_specs=pl.BlockSpec((RPW, 128), lambda j: (j, 0)),
    out_shape=jax.ShapeDtypeStruct((NQUAD, 128), jnp.float32),
)


def _proj_body(g_ref, m_ref, w_ref, b_ref, out_ref):
    g = g_ref[0]
    acc = lax.dot_general(
        w_ref[...], g,
        dimension_numbers=(((1,), (1,)), ((), ())),
        preferred_element_type=jnp.float32,
    )
    out_ref[0] = (acc + b_ref[...]) * m_ref[0]


_proj = pl.pallas_call(
    _proj_body,
    grid=(T,),
    in_specs=[
        pl.BlockSpec((1, B, DIM), lambda t: (t, 0, 0)),
        pl.BlockSpec((1, 1, B), lambda t: (t, 0, 0)),
        pl.BlockSpec((OUT_DIM, DIM), lambda t: (0, 0)),
        pl.BlockSpec((OUT_DIM, 1), lambda t: (0, 0)),
    ],
    out_specs=pl.BlockSpec((1, OUT_DIM, B), lambda t: (t, 0, 0)),
    out_shape=jax.ShapeDtypeStruct((T, OUT_DIM, B), jnp.float32),
)


def kernel(tokens, mask, embed_table, W, b):
    tokT = tokens.T.astype(jnp.int32)
    tT = embed_table.T
    table4 = _repack(tT, tT, tT, tT)
    inter = _sc_gather(tokT, table4)
    maskT = mask.T.astype(jnp.float32).reshape(T, 1, B)
    out = _proj(inter, maskT, W, b.reshape(OUT_DIM, 1))
    return jnp.transpose(out, (2, 0, 1)), mask


# speedup vs baseline: 1.7195x; 1.1015x over previous
import functools

import jax
import jax.numpy as jnp
from jax import lax
from jax.experimental import pallas as pl
from jax.experimental.pallas import tpu as pltpu
from jax.experimental.pallas import tpu_sc as plsc

DIM = 32
OUT_DIM = 64
B = 4096
T = 50
NROW = 1000000
NQUAD = 262144
OFF3 = 741376
K3_MIN = 786432

NC, NS = 2, 16
NW = NC * NS
NBLK = B // NW
L = 16

_sc_mesh = plsc.VectorSubcoreMesh(core_axis_name="c", subcore_axis_name="s")


@functools.partial(
    pl.kernel,
    out_type=jax.ShapeDtypeStruct((T, B // 4, 128), jnp.float32),
    mesh=_sc_mesh,
    scratch_types=[
        pltpu.VMEM((T, 4 * NBLK), jnp.int32),
        pltpu.VMEM((T, NBLK), jnp.int32),
        pltpu.VMEM((T, NBLK), jnp.int32),
        pltpu.VMEM((2, NBLK, 128), jnp.float32),
        pltpu.VMEM((2, NBLK // 4, 128), jnp.float32),
        pltpu.SemaphoreType.DMA,
        pltpu.SemaphoreType.DMA,
        pltpu.SemaphoreType.DMA,
        pltpu.SemaphoreType.DMA,
    ],
    compiler_params=pltpu.CompilerParams(use_tc_tiling_on_sc=True),
)
def _sc_gather(tokT_hbm, table_hbm, inter_hbm, tok_v, q_v, r_v, buf_v,
               slab_v, g0, g1, w0, w1):
    wid = lax.axis_index("s") * NC + lax.axis_index("c")
    m0 = wid * (NBLK // 4)

    for k in range(4):
        pltpu.sync_copy(
            tokT_hbm.at[:, pl.ds(k * (B // 4) + (wid // 4) * 128, 128)],
            tok_v.at[:, pl.ds(k * 128, 128)])
    sub = lax.mul(lax.rem(wid, 4), NBLK // 4)

    def _pre(t, _):
        for g in range(NBLK // L):
            k, l0 = divmod(g * L, NBLK // 4)
            tok = tok_v[t, pl.ds(k * 128 + sub + l0, L)]
            hi = tok >= K3_MIN
            pan = jnp.where(hi, 3, lax.shift_right_logical(tok, 18))
            sl = pl.ds(g * L, L)
            q_v[t, sl] = jnp.where(hi, tok - OFF3,
                                   lax.bitwise_and(tok, NQUAD - 1))
            r_v[t, sl] = lax.shift_left(pan, 5)
        return 0

    lax.fori_loop(0, T, _pre, 0, unroll=False)

    gsems = (g0, g1)
    wsems = (w0, w1)

    def _gather(i, par):
        return pltpu.make_async_copy(
            table_hbm.at[q_v.at[i]], buf_v.at[par], gsems[par])

    def _slab_copy(i, par):
        return pltpu.make_async_copy(
            slab_v.at[par], inter_hbm.at[i, pl.ds(m0, NBLK // 4), :],
            wsems[par])

    def _step(i, par):
        @pl.when(i + 1 < T)
        def _():
            _gather(i + 1, 1 - par).start()

        _gather(i, par).wait()

        @pl.when(i >= 2)
        def _():
            _slab_copy(i, par).wait()

        buf = buf_v.at[par]
        slab = slab_v.at[par]
        for g in range(NBLK // L):
            rsv = r_v[i, pl.ds(g * L, L)]
            for l in range(L):
                rs = rsv[l]
                j = g * L + l
                line = j & 31
                ko = (j >> 5) * DIM
                slab[line, pl.ds(ko, L)] = buf[j, pl.ds(rs, L)]
                slab[line, pl.ds(ko + L, L)] = buf[j, pl.ds(rs + L, L)]
        _slab_copy(i, par).start()
        return 0

    def _pair(k, _):
        _step(2 * k, 0)
        _step(2 * k + 1, 1)
        return 0

    _gather(0, 0).start()
    lax.fori_loop(0, T // 2, _pair, 0, unroll=False)

    _slab_copy(T - 2, 0).wait()
    _slab_copy(T - 1, 1).wait()


RPW = 4096
RPG = NQUAD // RPW
_KOFF = (0, NQUAD // RPW, 2 * NQUAD // RPW, OFF3 // RPW)


def _repack_body(x0_ref, x1_ref, x2_ref, x3_ref, out_ref):
    x4 = jnp.concatenate(
        [x0_ref[...], x1_ref[...], x2_ref[...], x3_ref[...]], axis=0)
    out_ref[...] = x4.T


_repack = pl.pallas_call(
    _repack_body,
    grid=(RPG,),
    in_specs=[
        pl.BlockSpec((DIM, RPW), lambda j, o=o: (0, o + j)) for o in _KOFF
    ],
    out_specs=pl.BlockSpec((RPW, 128), lambda j: (j, 0)),
    out_shape=jax.ShapeDtypeStruct((NQUAD, 128), jnp.float32),
)


def _proj_body(g_ref, m_ref, w_ref, b_ref, out_ref):
    g4 = g_ref[0]
    gt = g4.T
    m = m_ref[0]
    bb = b_ref[...]
    for k in range(4):
        acc = lax.dot_general(
            w_ref[...], gt[k * DIM:(k + 1) * DIM, :],
            dimension_numbers=(((1,), (0,)), ((), ())),
            preferred_element_type=jnp.float32,
        )
        nsl = slice(k * (B // 4), (k + 1) * (B // 4))
        out_ref[0, :, nsl] = (acc + bb) * m[:, nsl]


_proj = pl.pallas_call(
    _proj_body,
    grid=(T,),
    in_specs=[
        pl.BlockSpec((1, B // 4, 128), lambda t: (t, 0, 0)),
        pl.BlockSpec((1, 1, B), lambda t: (t, 0, 0)),
        pl.BlockSpec((OUT_DIM, DIM), lambda t: (0, 0)),
        pl.BlockSpec((OUT_DIM, 1), lambda t: (0, 0)),
    ],
    out_specs=pl.BlockSpec((1, OUT_DIM, B), lambda t: (t, 0, 0)),
    out_shape=jax.ShapeDtypeStruct((T, OUT_DIM, B), jnp.float32),
)


def kernel(tokens, mask, embed_table, W, b):
    tokT = tokens.T.astype(jnp.int32)
    tT = embed_table.T
    table4 = _repack(tT, tT, tT, tT)
    inter = _sc_gather(tokT, table4)
    maskT = mask.T.astype(jnp.float32).reshape(T, 1, B)
    out = _proj(inter, maskT, W, b.reshape(OUT_DIM, 1))
    return jnp.transpose(out, (2, 0, 1)), mask


# speedup vs baseline: 1.7470x; 1.0160x over previous
import functools

import jax
import jax.numpy as jnp
from jax import lax
from jax.experimental import pallas as pl
from jax.experimental.pallas import tpu as pltpu
from jax.experimental.pallas import tpu_sc as plsc

DIM = 32
OUT_DIM = 64
B = 4096
T = 50
NROW = 1000000
NQUAD = 262144
OFF3 = 741376
K3_MIN = 786432

NC, NS = 2, 16
NW = NC * NS
NBLK = B // NW
L = 16

_sc_mesh = plsc.VectorSubcoreMesh(core_axis_name="c", subcore_axis_name="s")


@functools.partial(
    pl.kernel,
    out_type=jax.ShapeDtypeStruct((T, B // 4, 128), jnp.float32),
    mesh=_sc_mesh,
    scratch_types=[
        pltpu.VMEM((T, 4 * NBLK), jnp.int32),
        pltpu.VMEM((T, NBLK), jnp.int32),
        pltpu.VMEM((T, NBLK), jnp.int32),
        pltpu.VMEM((4, NBLK, 128), jnp.float32),
        pltpu.VMEM((2, NBLK // 4, 128), jnp.float32),
        pltpu.SemaphoreType.DMA,
        pltpu.SemaphoreType.DMA,
        pltpu.SemaphoreType.DMA,
        pltpu.SemaphoreType.DMA,
        pltpu.SemaphoreType.DMA,
        pltpu.SemaphoreType.DMA,
    ],
    compiler_params=pltpu.CompilerParams(use_tc_tiling_on_sc=True),
)
def _sc_gather(tokT_hbm, table_hbm, inter_hbm, tok_v, q_v, r_v, buf_v,
               slab_v, g0, g1, g2, g3, w0, w1):
    wid = lax.axis_index("s") * NC + lax.axis_index("c")
    m0 = wid * (NBLK // 4)

    for k in range(4):
        pltpu.sync_copy(
            tokT_hbm.at[:, pl.ds(k * (B // 4) + (wid // 4) * 128, 128)],
            tok_v.at[:, pl.ds(k * 128, 128)])
    sub = lax.mul(lax.rem(wid, 4), NBLK // 4)

    def _pre(t, _):
        for g in range(NBLK // L):
            k, l0 = divmod(g * L, NBLK // 4)
            tok = tok_v[t, pl.ds(k * 128 + sub + l0, L)]
            hi = tok >= K3_MIN
            pan = jnp.where(hi, 3, lax.shift_right_logical(tok, 18))
            sl = pl.ds(g * L, L)
            q_v[t, sl] = jnp.where(hi, tok - OFF3,
                                   lax.bitwise_and(tok, NQUAD - 1))
            r_v[t, sl] = lax.shift_left(pan, 5)
        return 0

    lax.fori_loop(0, T, _pre, 0, unroll=False)

    gsems = (g0, g1, g2, g3)
    wsems = (w0, w1)

    def _gather(i, par):
        return pltpu.make_async_copy(
            table_hbm.at[q_v.at[i]], buf_v.at[par], gsems[par])

    def _slab_copy(i, par):
        return pltpu.make_async_copy(
            slab_v.at[par], inter_hbm.at[i, pl.ds(m0, NBLK // 4), :],
            wsems[par])

    def _step(i, p4, p2, fire_next, wait_slab):
        if fire_next:
            @pl.when(i + 3 < T)
            def _():
                _gather(i + 3, (p4 + 3) % 4).start()

        _gather(i, p4).wait()

        if wait_slab:
            @pl.when(i >= 2)
            def _():
                _slab_copy(i, p2).wait()

        buf = buf_v.at[p4]
        slab = slab_v.at[p2]
        for g in range(NBLK // L):
            rsv = r_v[i, pl.ds(g * L, L)]
            for l in range(L):
                rs = rsv[l]
                j = g * L + l
                line = j & 31
                ko = (j >> 5) * DIM
                slab[line, pl.ds(ko, L)] = buf[j, pl.ds(rs, L)]
                slab[line, pl.ds(ko + L, L)] = buf[j, pl.ds(rs + L, L)]
        _slab_copy(i, p2).start()
        return 0

    def _quad(k, _):
        i0 = 4 * k
        _step(i0, 0, 0, True, True)
        _step(i0 + 1, 1, 1, True, True)
        _step(i0 + 2, 2, 0, True, True)
        _step(i0 + 3, 3, 1, True, True)
        return 0

    _gather(0, 0).start()
    _gather(1, 1).start()
    _gather(2, 2).start()
    lax.fori_loop(0, (T - 2) // 4, _quad, 0, unroll=False)
    _step(T - 2, 0, 0, False, True)
    _step(T - 1, 1, 1, False, True)

    _slab_copy(T - 2, 0).wait()
    _slab_copy(T - 1, 1).wait()


RPW = 4096
RPG = NQUAD // RPW
_KOFF = (0, NQUAD // RPW, 2 * NQUAD // RPW, OFF3 // RPW)


def _repack_body(x0_ref, x1_ref, x2_ref, x3_ref, out_ref):
    x4 = jnp.concatenate(
        [x0_ref[...], x1_ref[...], x2_ref[...], x3_ref[...]], axis=0)
    out_ref[...] = x4.T


_repack = pl.pallas_call(
    _repack_body,
    grid=(RPG,),
    in_specs=[
        pl.BlockSpec((DIM, RPW), lambda j, o=o: (0, o + j)) for o in _KOFF
    ],
    out_specs=pl.BlockSpec((RPW, 128), lambda j: (j, 0)),
    out_shape=jax.ShapeDtypeStruct((NQUAD, 128), jnp.float32),
)


PT = 2


def _proj_body(g_ref, m_ref, w_ref, b_ref, out_ref):
    bb = b_ref[...]
    for tt in range(PT):
        g4 = g_ref[tt]
        gt = g4.T
        m = m_ref[tt]
        for k in range(4):
            acc = lax.dot_general(
                w_ref[...], gt[k * DIM:(k + 1) * DIM, :],
                dimension_numbers=(((1,), (0,)), ((), ())),
                preferred_element_type=jnp.float32,
            )
            nsl = slice(k * (B // 4), (k + 1) * (B // 4))
            out_ref[tt, :, nsl] = (acc + bb) * m[:, nsl]


_proj = pl.pallas_call(
    _proj_body,
    grid=(T // PT,),
    in_specs=[
        pl.BlockSpec((PT, B // 4, 128), lambda t: (t, 0, 0)),
        pl.BlockSpec((PT, 1, B), lambda t: (t, 0, 0)),
        pl.BlockSpec((OUT_DIM, DIM), lambda t: (0, 0)),
        pl.BlockSpec((OUT_DIM, 1), lambda t: (0, 0)),
    ],
    out_specs=pl.BlockSpec((PT, OUT_DIM, B), lambda t: (t, 0, 0)),
    out_shape=jax.ShapeDtypeStruct((T, OUT_DIM, B), jnp.float32),
)


def kernel(tokens, mask, embed_table, W, b):
    tokT = tokens.T.astype(jnp.int32)
    tT = embed_table.T
    table4 = _repack(tT, tT, tT, tT)
    inter = _sc_gather(tokT, table4)
    maskT = mask.T.astype(jnp.float32).reshape(T, 1, B)
    out = _proj(inter, maskT, W, b.reshape(OUT_DIM, 1))
    return jnp.transpose(out, (2, 0, 1)), mask


# speedup vs baseline: 1.8283x; 1.0465x over previous
import functools

import jax
import jax.numpy as jnp
from jax import lax
from jax.experimental import pallas as pl
from jax.experimental.pallas import tpu as pltpu
from jax.experimental.pallas import tpu_sc as plsc

DIM = 32
OUT_DIM = 64
B = 4096
T = 50
NROW = 1000000
NQUAD = 262144
OFF3 = 741376
K3_MIN = 786432

NC, NS = 2, 16
NW = NC * NS
NBLK = B // NW
L = 16

_sc_mesh = plsc.VectorSubcoreMesh(core_axis_name="c", subcore_axis_name="s")


@functools.partial(
    pl.kernel,
    out_type=jax.ShapeDtypeStruct((T, B // 4, 128), jnp.float32),
    mesh=_sc_mesh,
    scratch_types=[
        pltpu.VMEM((T, 4 * NBLK), jnp.int32),
        pltpu.VMEM((T, NBLK), jnp.int32),
        pltpu.VMEM((T, NBLK), jnp.int32),
        pltpu.VMEM((4, NBLK, 128), jnp.float32),
        pltpu.VMEM((2, NBLK // 4, 128), jnp.float32),
        pltpu.SemaphoreType.DMA,
        pltpu.SemaphoreType.DMA,
        pltpu.SemaphoreType.DMA,
        pltpu.SemaphoreType.DMA,
        pltpu.SemaphoreType.DMA,
        pltpu.SemaphoreType.DMA,
    ],
    compiler_params=pltpu.CompilerParams(use_tc_tiling_on_sc=True),
)
def _sc_gather(tokT_hbm, table_hbm, inter_hbm, tok_v, q_v, r_v, buf_v,
               slab_v, g0, g1, g2, g3, w0, w1):
    wid = lax.axis_index("s") * NC + lax.axis_index("c")
    m0 = wid * (NBLK // 4)

    for k in range(4):
        pltpu.sync_copy(
            tokT_hbm.at[:, pl.ds(k * (B // 4) + (wid // 4) * 128, 128)],
            tok_v.at[:, pl.ds(k * 128, 128)])
    sub = lax.mul(lax.rem(wid, 4), NBLK // 4)

    def _pre(t, _):
        for g in range(NBLK // L):
            k, l0 = divmod(g * L, NBLK // 4)
            tok = tok_v[t, pl.ds(k * 128 + sub + l0, L)]
            hi = tok >= K3_MIN
            pan = jnp.where(hi, 3, lax.shift_right_logical(tok, 18))
            sl = pl.ds(g * L, L)
            q_v[t, sl] = jnp.where(hi, tok - OFF3,
                                   lax.bitwise_and(tok, NQUAD - 1))
            r_v[t, sl] = lax.shift_left(pan, 5)
        return 0

    lax.fori_loop(0, T, _pre, 0, unroll=False)

    gsems = (g0, g1, g2, g3)
    wsems = (w0, w1)

    def _gather(i, par):
        return pltpu.make_async_copy(
            table_hbm.at[q_v.at[i]], buf_v.at[par], gsems[par])

    def _slab_copy(i, par):
        return pltpu.make_async_copy(
            slab_v.at[par], inter_hbm.at[i, pl.ds(m0, NBLK // 4), :],
            wsems[par])

    def _step(i, p4, p2, fire_next, wait_slab):
        if fire_next:
            @pl.when(i + 3 < T)
            def _():
                _gather(i + 3, (p4 + 3) % 4).start()

        _gather(i, p4).wait()

        if wait_slab:
            @pl.when(i >= 2)
            def _():
                _slab_copy(i, p2).wait()

        buf = buf_v.at[p4]
        slab = slab_v.at[p2]
        for g in range(NBLK // L):
            rsv = r_v[i, pl.ds(g * L, L)]
            for l in range(L):
                rs = rsv[l]
                j = g * L + l
                line = j & 31
                ko = (j >> 5) * DIM
                slab[line, pl.ds(ko, L)] = buf[j, pl.ds(rs, L)]
                slab[line, pl.ds(ko + L, L)] = buf[j, pl.ds(rs + L, L)]
        _slab_copy(i, p2).start()
        return 0

    def _quad(k, _):
        i0 = 4 * k
        _step(i0, 0, 0, True, True)
        _step(i0 + 1, 1, 1, True, True)
        _step(i0 + 2, 2, 0, True, True)
        _step(i0 + 3, 3, 1, True, True)
        return 0

    _gather(0, 0).start()
    _gather(1, 1).start()
    _gather(2, 2).start()
    lax.fori_loop(0, (T - 2) // 4, _quad, 0, unroll=False)
    _step(T - 2, 0, 0, False, True)
    _step(T - 1, 1, 1, False, True)

    _slab_copy(T - 2, 0).wait()
    _slab_copy(T - 1, 1).wait()


RPW = 4096
RPG = NQUAD // RPW
_KOFF = (0, NQUAD // RPW, 2 * NQUAD // RPW, OFF3 // RPW)


def _repack_body(x0_ref, x1_ref, x2_ref, x3_ref, out_ref):
    x4 = jnp.concatenate(
        [x0_ref[...], x1_ref[...], x2_ref[...], x3_ref[...]], axis=0)
    out_ref[...] = x4.T


_repack = pl.pallas_call(
    _repack_body,
    grid=(RPG,),
    in_specs=[
        pl.BlockSpec((DIM, RPW), lambda j, o=o: (0, o + j)) for o in _KOFF
    ],
    out_specs=pl.BlockSpec((RPW, 128), lambda j: (j, 0)),
    out_shape=jax.ShapeDtypeStruct((NQUAD, 128), jnp.float32),
)


PT = 5


def _proj_body(g_ref, m_ref, w_ref, b_ref, out_ref):
    bb = b_ref[...]
    for tt in range(PT):
        g4 = g_ref[tt]
        gt = g4.T
        m = m_ref[tt]
        for k in range(4):
            acc = lax.dot_general(
                w_ref[...], gt[k * DIM:(k + 1) * DIM, :],
                dimension_numbers=(((1,), (0,)), ((), ())),
                preferred_element_type=jnp.float32,
            )
            nsl = slice(k * (B // 4), (k + 1) * (B // 4))
            out_ref[tt, :, nsl] = (acc + bb) * m[:, nsl]


_proj = pl.pallas_call(
    _proj_body,
    grid=(T // PT,),
    in_specs=[
        pl.BlockSpec((PT, B // 4, 128), lambda t: (t, 0, 0)),
        pl.BlockSpec((PT, 1, B), lambda t: (t, 0, 0)),
        pl.BlockSpec((OUT_DIM, DIM), lambda t: (0, 0)),
        pl.BlockSpec((OUT_DIM, 1), lambda t: (0, 0)),
    ],
    out_specs=pl.BlockSpec((PT, OUT_DIM, B), lambda t: (t, 0, 0)),
    out_shape=jax.ShapeDtypeStruct((T, OUT_DIM, B), jnp.float32),
)


def kernel(tokens, mask, embed_table, W, b):
    tokT = tokens.T.astype(jnp.int32)
    tT = embed_table.T
    table4 = _repack(tT, tT, tT, tT)
    inter = _sc_gather(tokT, table4)
    maskT = mask.T.astype(jnp.float32).reshape(T, 1, B)
    out = _proj(inter, maskT, W, b.reshape(OUT_DIM, 1))
    return jnp.transpose(out, (2, 0, 1)), mask
